# baseline (speedup 1.0000x reference)
import jax
import jax.numpy as jnp
from jax import lax
from jax.experimental import pallas as pl
from jax.experimental.pallas import tpu as pltpu
from jax.experimental.pallas import tpu_sc as plsc

N_NODES = 10000
N_EDGES = 320000
D = 128

NC = 2
NS = 16
NW = NC * NS
CHUNK = 112
CPW = 90
EPW = CPW * CHUNK
E_PAD = NW * EPW
ROWS_PER_TILE = 624
LANES = 16


def _mm_sm_body(h_ref, w_ref, ef_ref, hc_ref, e_ref):
    w = w_ref[0] + w_ref[1]
    hc_ref[...] = jnp.dot(h_ref[...], w, preferred_element_type=jnp.float32)

    @pl.when(pl.program_id(0) == 0)
    def _softmax():
        x = ef_ref[...]
        ex = jnp.exp(x - jnp.max(x))
        e_ref[...] = ex / jnp.sum(ex)


def _matmul_softmax(h, weight, edge_feat):
    grid = 10
    rows = N_NODES // grid
    erows = N_EDGES // D
    return pl.pallas_call(
        _mm_sm_body,
        grid=(grid,),
        in_specs=[
            pl.BlockSpec((rows, D), lambda i: (i, 0)),
            pl.BlockSpec((2, D, D), lambda i: (0, 0, 0)),
            pl.BlockSpec((erows, D), lambda i: (0, 0)),
        ],
        out_specs=[
            pl.BlockSpec((rows, D), lambda i: (i, 0)),
            pl.BlockSpec((erows, D), lambda i: (0, 0)),
        ],
        out_shape=[
            jax.ShapeDtypeStruct((N_NODES, D), jnp.float32),
            jax.ShapeDtypeStruct((erows, D), jnp.float32),
        ],
    )(h, weight, edge_feat.reshape(erows, D))


def _comb_body(p_ref, b_ref, o_ref):
    o_ref[...] = p_ref[0] + p_ref[1] + b_ref[...]


def _combine(partials, bias):
    grid = 10
    rows = N_NODES // grid
    return pl.pallas_call(
        _comb_body,
        grid=(grid,),
        in_specs=[
            pl.BlockSpec((2, rows, D), lambda i: (0, i, 0)),
            pl.BlockSpec((1, D), lambda i: (0, 0)),
        ],
        out_specs=pl.BlockSpec((rows, D), lambda i: (i, 0)),
        out_shape=jax.ShapeDtypeStruct((N_NODES, D), jnp.float32),
    )(partials, bias.reshape(1, D))


def _sc_body(hc_hbm, src_hbm, dst_hbm, e_hbm, out_hbm,
             acc, is0, is1, is2, is3, is4, is5,
             id0, id1, id2, id3, id4, id5,
             ie0, ie1, ie2, ie3, ie4, ie5, g0, g1, g2,
             gsem0, gsem1, gsem2, ssem0, ssem1, ssem2, icsem0, icsem1):
    cid = lax.axis_index("c")
    sid = lax.axis_index("s")
    wid = cid * NS + sid
    gbuf = (g0, g1, g2)
    gsem = (gsem0, gsem1, gsem2)
    ssem = (ssem0, ssem1, ssem2)
    icsem = (icsem0, icsem1)
    isb = (is0, is1, is2, is3, is4, is5)
    idb = (id0, id1, id2, id3, id4, id5)
    ieb = (ie0, ie1, ie2, ie3, ie4, ie5)

    def _zero_row(i, _):
        for q in range(D // LANES):
            g0[i, pl.ds(q * LANES, LANES)] = jnp.zeros((LANES,), jnp.float32)
        return 0
    lax.fori_loop(0, CHUNK, _zero_row, 0)

    r0 = sid * ROWS_PER_TILE
    for k in range(5):
        pltpu.sync_copy(g0.at[pl.ds(0, CHUNK)],
                        acc.at[pl.ds(r0 + k * CHUNK, CHUNK)])
    pltpu.sync_copy(g0.at[pl.ds(0, 64)], acc.at[pl.ds(r0 + 560, 64)])

    @pl.when(sid == NS - 1)
    def _zero_tail():
        pltpu.sync_copy(g0.at[pl.ds(0, 16)],
                        acc.at[pl.ds(NS * ROWS_PER_TILE, 16)])
    plsc.subcore_barrier()

    for t in range(4):
        pltpu.sync_copy(src_hbm.at[wid, t], isb[t])
        pltpu.sync_copy(dst_hbm.at[wid, t], idb[t])
        pltpu.sync_copy(e_hbm.at[wid, t], ieb[t])
    pltpu.async_copy(hc_hbm.at[is0], g0, gsem0)
    pltpu.async_copy(hc_hbm.at[is1], g1, gsem1)

    def _turn(t, k):
        p = k % 3

        pltpu.make_async_copy(hc_hbm.at[isb[k]], gbuf[p], gsem[p]).wait()

        @plsc.parallel_loop(0, CHUNK, step=LANES)
        def _group(g):
            ev = ieb[k][pl.ds(g, LANES)]
            for j in range(LANES):
                s = ev[j]
                row = g + j
                for q in range(D // LANES):
                    sl = pl.ds(q * LANES, LANES)
                    gbuf[p][row, sl] = gbuf[p][row, sl] * s

        @pl.when(t >= 1)
        def _drain_scatter():
            pltpu.make_async_copy(gbuf[(p + 2) % 3], acc.at[idb[(k + 5) % 6]],
                                  ssem[(p + 2) % 3]).wait()

        @pl.when(jnp.logical_and(t >= 2, t + 2 < CPW))
        def _drain_idx():
            s2 = (k + 2) % 6
            pltpu.make_async_copy(src_hbm.at[wid, t + 2], isb[s2],
                                  icsem[k % 2]).wait()
            pltpu.make_async_copy(dst_hbm.at[wid, t + 2], idb[s2],
                                  icsem[k % 2]).wait()
            pltpu.make_async_copy(e_hbm.at[wid, t + 2], ieb[s2],
                                  icsem[k % 2]).wait()

        @pl.when(t + 2 < CPW)
        def _next_gather():
            pltpu.async_copy(hc_hbm.at[isb[(k + 2) % 6]], gbuf[(p + 2) % 3],
                             gsem[(p + 2) % 3])

        @pl.when(t + 4 < CPW)
        def _stage_idx():
            s4 = (k + 4) % 6
            pltpu.async_copy(src_hbm.at[wid, t + 4], isb[s4], icsem[k % 2])
            pltpu.async_copy(dst_hbm.at[wid, t + 4], idb[s4], icsem[k % 2])
            pltpu.async_copy(e_hbm.at[wid, t + 4], ieb[s4], icsem[k % 2])

        pltpu.async_copy(gbuf[p], acc.at[idb[k]], ssem[p], add=True)

    def _iter(i, _):
        for k in range(6):
            _turn(6 * i + k, k)
        return 0
    lax.fori_loop(0, CPW // 6, _iter, 0)

    pltpu.make_async_copy(gbuf[(CPW - 1) % 3], acc.at[idb[(CPW - 1) % 6]],
                          ssem[(CPW - 1) % 3]).wait()
    plsc.subcore_barrier()

    pltpu.sync_copy(acc.at[pl.ds(r0, ROWS_PER_TILE)],
                    out_hbm.at[cid, pl.ds(r0, ROWS_PER_TILE)])

    @pl.when(sid == NS - 1)
    def _write_tail():
        t0 = NS * ROWS_PER_TILE
        pltpu.sync_copy(acc.at[pl.ds(t0, N_NODES - NS * ROWS_PER_TILE)],
                        out_hbm.at[cid, pl.ds(t0, N_NODES - NS * ROWS_PER_TILE)])


_sc_scatter = pl.kernel(
    _sc_body,
    out_type=jax.ShapeDtypeStruct((NC, N_NODES, D), jnp.float32),
    mesh=plsc.VectorSubcoreMesh(core_axis_name="c", subcore_axis_name="s"),
    scratch_types=[
        pltpu.VMEM_SHARED((N_NODES, D), jnp.float32),
        pltpu.VMEM((CHUNK,), jnp.int32),
        pltpu.VMEM((CHUNK,), jnp.int32),
        pltpu.VMEM((CHUNK,), jnp.int32),
        pltpu.VMEM((CHUNK,), jnp.int32),
        pltpu.VMEM((CHUNK,), jnp.int32),
        pltpu.VMEM((CHUNK,), jnp.int32),
        pltpu.VMEM((CHUNK,), jnp.int32),
        pltpu.VMEM((CHUNK,), jnp.int32),
        pltpu.VMEM((CHUNK,), jnp.int32),
        pltpu.VMEM((CHUNK,), jnp.int32),
        pltpu.VMEM((CHUNK,), jnp.int32),
        pltpu.VMEM((CHUNK,), jnp.int32),
        pltpu.VMEM((CHUNK,), jnp.float32),
        pltpu.VMEM((CHUNK,), jnp.float32),
        pltpu.VMEM((CHUNK,), jnp.float32),
        pltpu.VMEM((CHUNK,), jnp.float32),
        pltpu.VMEM((CHUNK,), jnp.float32),
        pltpu.VMEM((CHUNK,), jnp.float32),
        pltpu.VMEM((CHUNK, D), jnp.float32),
        pltpu.VMEM((CHUNK, D), jnp.float32),
        pltpu.VMEM((CHUNK, D), jnp.float32),
        pltpu.SemaphoreType.DMA,
        pltpu.SemaphoreType.DMA,
        pltpu.SemaphoreType.DMA,
        pltpu.SemaphoreType.DMA,
        pltpu.SemaphoreType.DMA,
        pltpu.SemaphoreType.DMA,
        pltpu.SemaphoreType.DMA,
        pltpu.SemaphoreType.DMA,
    ],
)


def kernel(h, edge_index, edge_feat, weight, bias):
    hc, e = _matmul_softmax(h, weight, edge_feat)
    pad = E_PAD - N_EDGES
    src = jnp.pad(edge_index[0], (0, pad)).reshape(NW, CPW, CHUNK)
    dst = jnp.pad(edge_index[1], (0, pad)).reshape(NW, CPW, CHUNK)
    ep = jnp.pad(e.reshape(N_EDGES), (0, pad)).reshape(NW, CPW, CHUNK)
    partials = _sc_scatter(hc, src, dst, ep)
    return _combine(partials, bias)
